# baseline (speedup 1.0000x reference)
import functools

import jax
import jax.numpy as jnp
from jax.experimental import pallas as pl
from jax.experimental.pallas import tpu as pltpu

_D = 64
_NE = 1024
_K = 64
_R = 128
_NROW = 8192


def _stage(v, ix, j, asc, ln):
    length = v.shape[1]
    up = (ln & j) == 0
    vp = jnp.where(up, pltpu.roll(v, length - j, axis=1), pltpu.roll(v, j, axis=1))
    ip = jnp.where(up, pltpu.roll(ix, length - j, axis=1), pltpu.roll(ix, j, axis=1))
    w = (v > vp) | ((v == vp) & (ix < ip))
    take_mine = jnp.logical_not(w ^ (up ^ asc))
    return jnp.where(take_mine, v, vp), jnp.where(take_mine, ix, ip)


def _top64(p, iota_e):
    v, ix = p, iota_e
    k = 2
    nstage = 1
    while k <= _K:
        ln = jax.lax.broadcasted_iota(jnp.int32, (1, v.shape[1]), 1)
        asc = (ln // k) % 2 == 1

        def substage(t, carry, k=k, asc=asc, ln=ln):
            vv, ii = carry
            j = jax.lax.shift_right_logical(jnp.int32(k), t + 1)
            return _stage(vv, ii, j, asc, ln)

        v, ix = jax.lax.fori_loop(0, nstage, substage, (v, ix))
        k *= 2
        nstage += 1
    length = p.shape[1]
    while length > _K:
        nch = length // 128
        va = jnp.concatenate([v[:, c * 128:c * 128 + 64] for c in range(nch)], axis=1) if nch > 1 else v[:, :64]
        vb = jnp.concatenate([v[:, c * 128 + 64:(c + 1) * 128] for c in range(nch)], axis=1) if nch > 1 else v[:, 64:]
        ia = jnp.concatenate([ix[:, c * 128:c * 128 + 64] for c in range(nch)], axis=1) if nch > 1 else ix[:, :64]
        ib = jnp.concatenate([ix[:, c * 128 + 64:(c + 1) * 128] for c in range(nch)], axis=1) if nch > 1 else ix[:, 64:]
        w = (va > vb) | ((va == vb) & (ia < ib))
        v = jnp.where(w, va, vb)
        ix = jnp.where(w, ia, ib)
        length //= 2
        ln = jax.lax.broadcasted_iota(jnp.int32, (1, length), 1)
        if length > _K:
            asc = (ln // _K) % 2 == 1
        else:
            asc = jnp.zeros((1, length), jnp.bool_)

        def cleanstage(t, carry, asc=asc, ln=ln):
            vv, ii = carry
            j = jax.lax.shift_right_logical(jnp.int32(_K), t + 1)
            return _stage(vv, ii, j, asc, ln)

        v, ix = jax.lax.fori_loop(0, 6, cleanstage, (v, ix))
    return v, ix


def _body(x_ref, g_ref, e_ref, dist_ref, qhc_ref, qlc_ref, ihc_ref, ilc_ref):
    e = e_ref[...]
    x = x_ref[...]
    iota_e = jax.lax.broadcasted_iota(jnp.int32, (_R, _NE), 1)
    iota_k = jax.lax.broadcasted_iota(jnp.int32, (_R, _K), 1)

    r = jnp.sum(x * x, axis=1, keepdims=True)
    c = jnp.sum(e * e, axis=0, keepdims=True)
    m2 = jnp.dot(2.0 * x, e, preferred_element_type=jnp.float32)
    dist = (r - m2) + c
    dist_ref[...] = dist

    nd = -dist
    mv = jnp.max(nd, axis=1, keepdims=True)
    ihc = jnp.min(jnp.where(nd == mv, iota_e, _NE), axis=1, keepdims=True)
    ihc_ref[...] = ihc

    p = 1.0 / dist
    ps, sel = _top64(p, iota_e)

    psn = ps / jnp.sum(ps, axis=1, keepdims=True)
    y = jnp.log(jnp.clip(psn, 1e-30, None)) + g_ref[...]
    ymax = jnp.max(y, axis=1, keepdims=True)
    samp = jnp.min(jnp.where(y == ymax, iota_k, _K), axis=1, keepdims=True)
    ilc = jnp.sum(jnp.where(iota_k == samp, sel, 0), axis=1, keepdims=True)
    ilc_ref[...] = ilc

    oh_hc = (iota_e == ihc).astype(jnp.float32)
    oh_lc = (iota_e == ilc).astype(jnp.float32)
    dn = (((1,), (1,)), ((), ()))
    qhc_ref[...] = jax.lax.dot_general(
        oh_hc, e, dn, precision=jax.lax.Precision.HIGHEST,
        preferred_element_type=jnp.float32)
    qlc_ref[...] = jax.lax.dot_general(
        oh_lc, e, dn, precision=jax.lax.Precision.HIGHEST,
        preferred_element_type=jnp.float32)


@functools.partial(jax.jit, static_argnames=())
def _run(flat_hr, flat_lr, embed_lr, g_hr, g_lr):
    nb = _NROW // _R
    row_blk = lambda i: (i, 0)
    out_shapes = []
    out_specs = []
    for _ in range(2):
        out_shapes += [
            jax.ShapeDtypeStruct((_NROW, _NE), jnp.float32),
            jax.ShapeDtypeStruct((_NROW, _D), jnp.float32),
            jax.ShapeDtypeStruct((_NROW, _D), jnp.float32),
            jax.ShapeDtypeStruct((_NROW, 1), jnp.int32),
            jax.ShapeDtypeStruct((_NROW, 1), jnp.int32),
        ]
        out_specs += [
            pl.BlockSpec((_R, _NE), row_blk),
            pl.BlockSpec((_R, _D), row_blk),
            pl.BlockSpec((_R, _D), row_blk),
            pl.BlockSpec((_R, 1), row_blk),
            pl.BlockSpec((_R, 1), row_blk),
        ]

    def fused(xh_ref, xl_ref, gh_ref, gl_ref, e_ref,
              dh_ref, qhch_ref, qlch_ref, ihch_ref, ilch_ref,
              dl_ref, qhcl_ref, qlcl_ref, ihcl_ref, ilcl_ref):
        _body(xh_ref, gh_ref, e_ref, dh_ref, qhch_ref, qlch_ref, ihch_ref, ilch_ref)
        _body(xl_ref, gl_ref, e_ref, dl_ref, qhcl_ref, qlcl_ref, ihcl_ref, ilcl_ref)

    return pl.pallas_call(
        fused,
        grid=(nb,),
        in_specs=[
            pl.BlockSpec((_R, _D), row_blk),
            pl.BlockSpec((_R, _D), row_blk),
            pl.BlockSpec((_R, _K), row_blk),
            pl.BlockSpec((_R, _K), row_blk),
            pl.BlockSpec((_D, _NE), lambda i: (0, 0)),
        ],
        out_specs=out_specs,
        out_shape=out_shapes,
        compiler_params=pltpu.CompilerParams(
            dimension_semantics=("arbitrary",)),
    )(flat_hr, flat_lr, g_hr, g_lr, embed_lr)


def kernel(input_hr, input_lr, embed_lr):
    flat_hr = input_hr.reshape(-1, _D)
    flat_lr = input_lr.reshape(-1, _D)

    skey = jax.random.key(42)
    k1, k2 = jax.random.split(skey)
    g_lr = jax.random.gumbel(k1, (_NROW, _K), jnp.float32)
    g_hr = jax.random.gumbel(k2, (_NROW, _K), jnp.float32)

    (dist_hr, q_hc_hr, q_lc_hr, i_hc_hr, i_lc_hr,
     dist_lr, q_hc_lr, q_lc_lr, i_hc_lr, i_lc_lr) = _run(
        flat_hr, flat_lr, embed_lr, g_hr, g_lr)

    shp = input_hr.shape
    ind_shp = shp[:-1]

    def finish(q, inp):
        q = q.reshape(shp)
        d = q - inp
        return inp + d, jnp.mean(d * d)

    quantize_hc_hr, diff_hc_hr = finish(q_hc_hr, input_hr)
    quantize_hc_lr, diff_hc_lr = finish(q_hc_lr, input_lr)
    quantize_lc_hr, diff_lc_hr = finish(q_lc_hr, input_hr)
    quantize_lc_lr, diff_lc_lr = finish(q_lc_lr, input_lr)

    embed_ind_hc_hr = i_hc_hr.reshape(ind_shp)
    embed_ind_hc_lr = i_hc_lr.reshape(ind_shp)
    embed_ind_lc_hr = i_lc_hr.reshape(ind_shp)
    embed_ind_lc_lr = i_lc_lr.reshape(ind_shp)

    return (quantize_hc_hr, quantize_hc_lr, quantize_lc_hr, quantize_lc_lr,
            diff_hc_hr, diff_hc_lr, diff_lc_hr, diff_lc_lr,
            embed_ind_hc_hr, embed_ind_hc_lr, embed_ind_lc_hr, embed_ind_lc_lr,
            dist_hr, dist_lr, dist_hr, dist_lr)


# speedup vs baseline: 1.9811x; 1.9811x over previous
import functools

import jax
import jax.numpy as jnp
from jax.experimental import pallas as pl
from jax.experimental.pallas import tpu as pltpu

_D = 64
_NE = 1024
_K = 64
_R = 256
_NROW = 8192


def _body(x_ref, g_ref, e_ref, dist_ref, qhc_ref, qlc_ref, ihc_ref, ilc_ref):
    e = e_ref[...]
    x = x_ref[...]
    iota_e = jax.lax.broadcasted_iota(jnp.int32, (_R, _NE), 1)
    iota_k = jax.lax.broadcasted_iota(jnp.int32, (_R, _K), 1)

    r = jnp.sum(x * x, axis=1, keepdims=True)
    c = jnp.sum(e * e, axis=0, keepdims=True)
    m2 = jnp.dot(2.0 * x, e, preferred_element_type=jnp.float32)
    dist = (r - m2) + c
    dist_ref[...] = dist

    nd = -dist
    mv = jnp.max(nd, axis=1, keepdims=True)
    ihc = jnp.min(jnp.where(nd == mv, iota_e, _NE), axis=1, keepdims=True)
    ihc_ref[...] = ihc

    p = 1.0 / dist

    def step(k, carry):
        pm, sel, ps = carry
        v = jnp.max(pm, axis=1, keepdims=True)
        t = jnp.where(pm == v, iota_e, _NE)
        idx = jnp.min(t, axis=1, keepdims=True)
        pm = jnp.where(t == idx, -jnp.inf, pm)
        sel = jnp.where(iota_k == k, idx, sel)
        ps = jnp.where(iota_k == k, v, ps)
        return pm, sel, ps

    _, sel, ps = jax.lax.fori_loop(
        0, _K, step,
        (p, jnp.zeros((_R, _K), jnp.int32), jnp.zeros((_R, _K), jnp.float32)))

    psn = ps / jnp.sum(ps, axis=1, keepdims=True)
    y = jnp.log(jnp.clip(psn, 1e-30, None)) + g_ref[...]
    ymax = jnp.max(y, axis=1, keepdims=True)
    samp = jnp.min(jnp.where(y == ymax, iota_k, _K), axis=1, keepdims=True)
    ilc = jnp.sum(jnp.where(iota_k == samp, sel, 0), axis=1, keepdims=True)
    ilc_ref[...] = ilc

    oh_hc = (iota_e == ihc).astype(jnp.float32)
    oh_lc = (iota_e == ilc).astype(jnp.float32)
    dn = (((1,), (1,)), ((), ()))
    qhc_ref[...] = jax.lax.dot_general(
        oh_hc, e, dn, precision=jax.lax.Precision.HIGHEST,
        preferred_element_type=jnp.float32)
    qlc_ref[...] = jax.lax.dot_general(
        oh_lc, e, dn, precision=jax.lax.Precision.HIGHEST,
        preferred_element_type=jnp.float32)


@functools.partial(jax.jit, static_argnames=())
def _run(flat_hr, flat_lr, embed_lr, g_hr, g_lr):
    nb = _NROW // _R
    row_blk = lambda i: (i, 0)
    out_shapes = []
    out_specs = []
    for _ in range(2):
        out_shapes += [
            jax.ShapeDtypeStruct((_NROW, _NE), jnp.float32),
            jax.ShapeDtypeStruct((_NROW, _D), jnp.float32),
            jax.ShapeDtypeStruct((_NROW, _D), jnp.float32),
            jax.ShapeDtypeStruct((_NROW, 1), jnp.int32),
            jax.ShapeDtypeStruct((_NROW, 1), jnp.int32),
        ]
        out_specs += [
            pl.BlockSpec((_R, _NE), row_blk),
            pl.BlockSpec((_R, _D), row_blk),
            pl.BlockSpec((_R, _D), row_blk),
            pl.BlockSpec((_R, 1), row_blk),
            pl.BlockSpec((_R, 1), row_blk),
        ]

    def fused(xh_ref, xl_ref, gh_ref, gl_ref, e_ref,
              dh_ref, qhch_ref, qlch_ref, ihch_ref, ilch_ref,
              dl_ref, qhcl_ref, qlcl_ref, ihcl_ref, ilcl_ref):
        _body(xh_ref, gh_ref, e_ref, dh_ref, qhch_ref, qlch_ref, ihch_ref, ilch_ref)
        _body(xl_ref, gl_ref, e_ref, dl_ref, qhcl_ref, qlcl_ref, ihcl_ref, ilcl_ref)

    return pl.pallas_call(
        fused,
        grid=(nb,),
        in_specs=[
            pl.BlockSpec((_R, _D), row_blk),
            pl.BlockSpec((_R, _D), row_blk),
            pl.BlockSpec((_R, _K), row_blk),
            pl.BlockSpec((_R, _K), row_blk),
            pl.BlockSpec((_D, _NE), lambda i: (0, 0)),
        ],
        out_specs=out_specs,
        out_shape=out_shapes,
        compiler_params=pltpu.CompilerParams(
            dimension_semantics=("arbitrary",)),
    )(flat_hr, flat_lr, g_hr, g_lr, embed_lr)


def kernel(input_hr, input_lr, embed_lr):
    flat_hr = input_hr.reshape(-1, _D)
    flat_lr = input_lr.reshape(-1, _D)

    skey = jax.random.key(42)
    k1, k2 = jax.random.split(skey)
    g_lr = jax.random.gumbel(k1, (_NROW, _K), jnp.float32)
    g_hr = jax.random.gumbel(k2, (_NROW, _K), jnp.float32)

    (dist_hr, q_hc_hr, q_lc_hr, i_hc_hr, i_lc_hr,
     dist_lr, q_hc_lr, q_lc_lr, i_hc_lr, i_lc_lr) = _run(
        flat_hr, flat_lr, embed_lr, g_hr, g_lr)

    shp = input_hr.shape
    ind_shp = shp[:-1]

    def finish(q, inp):
        q = q.reshape(shp)
        d = q - inp
        return inp + d, jnp.mean(d * d)

    quantize_hc_hr, diff_hc_hr = finish(q_hc_hr, input_hr)
    quantize_hc_lr, diff_hc_lr = finish(q_hc_lr, input_lr)
    quantize_lc_hr, diff_lc_hr = finish(q_lc_hr, input_hr)
    quantize_lc_lr, diff_lc_lr = finish(q_lc_lr, input_lr)

    embed_ind_hc_hr = i_hc_hr.reshape(ind_shp)
    embed_ind_hc_lr = i_hc_lr.reshape(ind_shp)
    embed_ind_lc_hr = i_lc_hr.reshape(ind_shp)
    embed_ind_lc_lr = i_lc_lr.reshape(ind_shp)

    return (quantize_hc_hr, quantize_hc_lr, quantize_lc_hr, quantize_lc_lr,
            diff_hc_hr, diff_hc_lr, diff_lc_hr, diff_lc_lr,
            embed_ind_hc_hr, embed_ind_hc_lr, embed_ind_lc_hr, embed_ind_lc_lr,
            dist_hr, dist_lr, dist_hr, dist_lr)


# speedup vs baseline: 2.1400x; 1.0802x over previous
import functools

import jax
import jax.numpy as jnp
from jax import lax
from jax.experimental import pallas as pl
from jax.experimental.pallas import tpu as pltpu
from jax.experimental.pallas import tpu_sc as plsc

_D = 64
_NE = 1024
_K = 64
_R = 256
_NROW = 8192


_NW = 32
_BG = 4 * _NROW
_BPW = _BG // _NW
_CH = 128


def _sc_gather(table, idx2d):
    mesh = plsc.VectorSubcoreMesh(core_axis_name="c", subcore_axis_name="s")
    nch = _BPW // _CH

    @functools.partial(
        pl.kernel, mesh=mesh,
        out_type=jax.ShapeDtypeStruct((_BG, 128), jnp.float32),
        scratch_types=[
            pltpu.VMEM((nch, _CH), jnp.int32),
            pltpu.VMEM((_CH, 128), jnp.float32),
            pltpu.SemaphoreType.DMA,
        ],
    )
    def k(table_hbm, idx_hbm, out_hbm, idx_v, rows_v, sem):
        wid = lax.axis_index("s") * 2 + lax.axis_index("c")
        pltpu.sync_copy(idx_hbm.at[pl.ds(wid * nch, nch)], idx_v)
        for ci in range(nch):
            pltpu.async_copy(table_hbm.at[idx_v.at[ci]], rows_v, sem).wait()
            pltpu.sync_copy(rows_v, out_hbm.at[pl.ds(wid * _BPW + ci * _CH, _CH)])

    return k(table, idx2d)


def _body(x_ref, g_ref, e_ref, dist_ref, ihc_ref, ilc_ref):
    e = e_ref[...]
    x = x_ref[...]
    iota_e = jax.lax.broadcasted_iota(jnp.int32, (_R, _NE), 1)
    iota_k = jax.lax.broadcasted_iota(jnp.int32, (_R, _K), 1)

    r = jnp.sum(x * x, axis=1, keepdims=True)
    c = jnp.sum(e * e, axis=0, keepdims=True)
    m2 = jnp.dot(2.0 * x, e, preferred_element_type=jnp.float32)
    dist = (r - m2) + c
    dist_ref[...] = dist

    nd = -dist
    mv = jnp.max(nd, axis=1, keepdims=True)
    ihc = jnp.min(jnp.where(nd == mv, iota_e, _NE), axis=1, keepdims=True)
    ihc_ref[...] = ihc

    p = 1.0 / dist

    def step(k, carry):
        pm, sel, ps = carry
        v = jnp.max(pm, axis=1, keepdims=True)
        idx = jnp.min(jnp.where(pm == v, iota_e, _NE), axis=1, keepdims=True)
        pm = jnp.where(iota_e == idx, -jnp.inf, pm)
        sel = jnp.where(iota_k == k, idx, sel)
        ps = jnp.where(iota_k == k, v, ps)
        return pm, sel, ps

    _, sel, ps = jax.lax.fori_loop(
        0, _K, step,
        (p, jnp.zeros((_R, _K), jnp.int32), jnp.zeros((_R, _K), jnp.float32)))

    psn = ps / jnp.sum(ps, axis=1, keepdims=True)
    y = jnp.log(jnp.clip(psn, 1e-30, None)) + g_ref[...]
    ymax = jnp.max(y, axis=1, keepdims=True)
    samp = jnp.min(jnp.where(y == ymax, iota_k, _K), axis=1, keepdims=True)
    ilc = jnp.sum(jnp.where(iota_k == samp, sel, 0), axis=1, keepdims=True)
    ilc_ref[...] = ilc


@functools.partial(jax.jit, static_argnames=())
def _run(flat_hr, flat_lr, embed_lr, g_hr, g_lr):
    nb = _NROW // _R
    row_blk = lambda i: (i, 0)
    out_shapes = []
    out_specs = []
    for _ in range(2):
        out_shapes += [
            jax.ShapeDtypeStruct((_NROW, _NE), jnp.float32),
            jax.ShapeDtypeStruct((_NROW, 1), jnp.int32),
            jax.ShapeDtypeStruct((_NROW, 1), jnp.int32),
        ]
        out_specs += [
            pl.BlockSpec((_R, _NE), row_blk),
            pl.BlockSpec((_R, 1), row_blk),
            pl.BlockSpec((_R, 1), row_blk),
        ]

    def fused(xh_ref, xl_ref, gh_ref, gl_ref, e_ref,
              dh_ref, ihch_ref, ilch_ref,
              dl_ref, ihcl_ref, ilcl_ref):
        _body(xh_ref, gh_ref, e_ref, dh_ref, ihch_ref, ilch_ref)
        _body(xl_ref, gl_ref, e_ref, dl_ref, ihcl_ref, ilcl_ref)

    return pl.pallas_call(
        fused,
        grid=(nb,),
        in_specs=[
            pl.BlockSpec((_R, _D), row_blk),
            pl.BlockSpec((_R, _D), row_blk),
            pl.BlockSpec((_R, _K), row_blk),
            pl.BlockSpec((_R, _K), row_blk),
            pl.BlockSpec((_D, _NE), lambda i: (0, 0)),
        ],
        out_specs=out_specs,
        out_shape=out_shapes,
        compiler_params=pltpu.CompilerParams(
            dimension_semantics=("arbitrary",)),
    )(flat_hr, flat_lr, g_hr, g_lr, embed_lr)


def kernel(input_hr, input_lr, embed_lr):
    flat_hr = input_hr.reshape(-1, _D)
    flat_lr = input_lr.reshape(-1, _D)

    skey = jax.random.key(42)
    k1, k2 = jax.random.split(skey)
    g_lr = jax.random.gumbel(k1, (_NROW, _K), jnp.float32)
    g_hr = jax.random.gumbel(k2, (_NROW, _K), jnp.float32)

    (dist_hr, i_hc_hr, i_lc_hr,
     dist_lr, i_hc_lr, i_lc_lr) = _run(
        flat_hr, flat_lr, embed_lr, g_hr, g_lr)

    table = jnp.pad(embed_lr.T, ((0, 0), (0, 128 - _D)))
    idx2d = jnp.concatenate(
        [i_hc_hr, i_hc_lr, i_lc_hr, i_lc_lr], axis=0).reshape(_BG // _CH, _CH)
    q_all = _sc_gather(table, idx2d)[:, :_D]
    q_hc_hr = q_all[0 * _NROW:1 * _NROW]
    q_hc_lr = q_all[1 * _NROW:2 * _NROW]
    q_lc_hr = q_all[2 * _NROW:3 * _NROW]
    q_lc_lr = q_all[3 * _NROW:4 * _NROW]

    shp = input_hr.shape
    ind_shp = shp[:-1]

    def finish(q, inp):
        q = q.reshape(shp)
        d = q - inp
        return inp + d, jnp.mean(d * d)

    quantize_hc_hr, diff_hc_hr = finish(q_hc_hr, input_hr)
    quantize_hc_lr, diff_hc_lr = finish(q_hc_lr, input_lr)
    quantize_lc_hr, diff_lc_hr = finish(q_lc_hr, input_hr)
    quantize_lc_lr, diff_lc_lr = finish(q_lc_lr, input_lr)

    embed_ind_hc_hr = i_hc_hr.reshape(ind_shp)
    embed_ind_hc_lr = i_hc_lr.reshape(ind_shp)
    embed_ind_lc_hr = i_lc_hr.reshape(ind_shp)
    embed_ind_lc_lr = i_lc_lr.reshape(ind_shp)

    return (quantize_hc_hr, quantize_hc_lr, quantize_lc_hr, quantize_lc_lr,
            diff_hc_hr, diff_hc_lr, diff_lc_hr, diff_lc_lr,
            embed_ind_hc_hr, embed_ind_hc_lr, embed_ind_lc_hr, embed_ind_lc_lr,
            dist_hr, dist_lr, dist_hr, dist_lr)


# speedup vs baseline: 2.3791x; 1.1117x over previous
import functools

import jax
import jax.numpy as jnp
from jax import lax
from jax.experimental import pallas as pl
from jax.experimental.pallas import tpu as pltpu
from jax.experimental.pallas import tpu_sc as plsc

_D = 64
_NE = 1024
_K = 64
_R = 512
_NROW = 8192


_NW = 32
_BG = 4 * _NROW
_BPW = _BG // _NW
_CH = 128


def _sc_gather(table, idx2d):
    mesh = plsc.VectorSubcoreMesh(core_axis_name="c", subcore_axis_name="s")
    nch = _BPW // _CH

    @functools.partial(
        pl.kernel, mesh=mesh,
        out_type=jax.ShapeDtypeStruct((_BG, 128), jnp.float32),
        scratch_types=[
            pltpu.VMEM((nch, _CH), jnp.int32),
            pltpu.VMEM((_CH, 128), jnp.float32),
            pltpu.SemaphoreType.DMA,
        ],
    )
    def k(table_hbm, idx_hbm, out_hbm, idx_v, rows_v, sem):
        wid = lax.axis_index("s") * 2 + lax.axis_index("c")
        pltpu.sync_copy(idx_hbm.at[pl.ds(wid * nch, nch)], idx_v)
        for ci in range(nch):
            pltpu.async_copy(table_hbm.at[idx_v.at[ci]], rows_v, sem).wait()
            pltpu.sync_copy(rows_v, out_hbm.at[pl.ds(wid * _BPW + ci * _CH, _CH)])

    return k(table, idx2d)


def _body(x_ref, g_ref, e_ref, dist_ref, ihc_ref, ilc_ref):
    e = e_ref[...]
    x = x_ref[...]
    iota_e = jax.lax.broadcasted_iota(jnp.int32, (_R, _NE), 1)
    iota_k = jax.lax.broadcasted_iota(jnp.int32, (_R, _K), 1)

    r = jnp.sum(x * x, axis=1, keepdims=True)
    c = jnp.sum(e * e, axis=0, keepdims=True)
    m2 = jnp.dot(2.0 * x, e, preferred_element_type=jnp.float32)
    dist = (r - m2) + c
    dist_ref[...] = dist

    nd = -dist
    mv = jnp.max(nd, axis=1, keepdims=True)
    ihc = jnp.min(jnp.where(nd == mv, iota_e, _NE), axis=1, keepdims=True)
    ihc_ref[...] = ihc

    p = 1.0 / dist

    def step(k, carry):
        pm, sel, ps = carry
        v = jnp.max(pm, axis=1, keepdims=True)
        idx = jnp.min(jnp.where(pm == v, iota_e, _NE), axis=1, keepdims=True)
        pm = jnp.where(iota_e == idx, -jnp.inf, pm)
        sel = jnp.where(iota_k == k, idx, sel)
        ps = jnp.where(iota_k == k, v, ps)
        return pm, sel, ps

    _, sel, ps = jax.lax.fori_loop(
        0, _K, step,
        (p, jnp.zeros((_R, _K), jnp.int32), jnp.zeros((_R, _K), jnp.float32)))

    psn = ps / jnp.sum(ps, axis=1, keepdims=True)
    y = jnp.log(jnp.clip(psn, 1e-30, None)) + g_ref[...]
    ymax = jnp.max(y, axis=1, keepdims=True)
    samp = jnp.min(jnp.where(y == ymax, iota_k, _K), axis=1, keepdims=True)
    ilc = jnp.sum(jnp.where(iota_k == samp, sel, 0), axis=1, keepdims=True)
    ilc_ref[...] = ilc


@functools.partial(jax.jit, static_argnames=())
def _run(flat_hr, flat_lr, embed_lr, g_hr, g_lr):
    nb = _NROW // _R
    row_blk = lambda i: (i, 0)
    out_shapes = []
    out_specs = []
    for _ in range(2):
        out_shapes += [
            jax.ShapeDtypeStruct((_NROW, _NE), jnp.float32),
            jax.ShapeDtypeStruct((_NROW, 1), jnp.int32),
            jax.ShapeDtypeStruct((_NROW, 1), jnp.int32),
        ]
        out_specs += [
            pl.BlockSpec((_R, _NE), row_blk),
            pl.BlockSpec((_R, 1), row_blk),
            pl.BlockSpec((_R, 1), row_blk),
        ]

    def fused(xh_ref, xl_ref, gh_ref, gl_ref, e_ref,
              dh_ref, ihch_ref, ilch_ref,
              dl_ref, ihcl_ref, ilcl_ref):
        _body(xh_ref, gh_ref, e_ref, dh_ref, ihch_ref, ilch_ref)
        _body(xl_ref, gl_ref, e_ref, dl_ref, ihcl_ref, ilcl_ref)

    return pl.pallas_call(
        fused,
        grid=(nb,),
        in_specs=[
            pl.BlockSpec((_R, _D), row_blk),
            pl.BlockSpec((_R, _D), row_blk),
            pl.BlockSpec((_R, _K), row_blk),
            pl.BlockSpec((_R, _K), row_blk),
            pl.BlockSpec((_D, _NE), lambda i: (0, 0)),
        ],
        out_specs=out_specs,
        out_shape=out_shapes,
        compiler_params=pltpu.CompilerParams(
            dimension_semantics=("arbitrary",)),
    )(flat_hr, flat_lr, g_hr, g_lr, embed_lr)


def kernel(input_hr, input_lr, embed_lr):
    flat_hr = input_hr.reshape(-1, _D)
    flat_lr = input_lr.reshape(-1, _D)

    skey = jax.random.key(42)
    k1, k2 = jax.random.split(skey)
    g_lr = jax.random.gumbel(k1, (_NROW, _K), jnp.float32)
    g_hr = jax.random.gumbel(k2, (_NROW, _K), jnp.float32)

    (dist_hr, i_hc_hr, i_lc_hr,
     dist_lr, i_hc_lr, i_lc_lr) = _run(
        flat_hr, flat_lr, embed_lr, g_hr, g_lr)

    table = jnp.pad(embed_lr.T, ((0, 0), (0, 128 - _D)))
    idx2d = jnp.concatenate(
        [i_hc_hr, i_hc_lr, i_lc_hr, i_lc_lr], axis=0).reshape(_BG // _CH, _CH)
    q_all = _sc_gather(table, idx2d)[:, :_D]
    q_hc_hr = q_all[0 * _NROW:1 * _NROW]
    q_hc_lr = q_all[1 * _NROW:2 * _NROW]
    q_lc_hr = q_all[2 * _NROW:3 * _NROW]
    q_lc_lr = q_all[3 * _NROW:4 * _NROW]

    shp = input_hr.shape
    ind_shp = shp[:-1]

    def finish(q, inp):
        q = q.reshape(shp)
        d = q - inp
        return inp + d, jnp.mean(d * d)

    quantize_hc_hr, diff_hc_hr = finish(q_hc_hr, input_hr)
    quantize_hc_lr, diff_hc_lr = finish(q_hc_lr, input_lr)
    quantize_lc_hr, diff_lc_hr = finish(q_lc_hr, input_hr)
    quantize_lc_lr, diff_lc_lr = finish(q_lc_lr, input_lr)

    embed_ind_hc_hr = i_hc_hr.reshape(ind_shp)
    embed_ind_hc_lr = i_hc_lr.reshape(ind_shp)
    embed_ind_lc_hr = i_lc_hr.reshape(ind_shp)
    embed_ind_lc_lr = i_lc_lr.reshape(ind_shp)

    return (quantize_hc_hr, quantize_hc_lr, quantize_lc_hr, quantize_lc_lr,
            diff_hc_hr, diff_hc_lr, diff_lc_hr, diff_lc_lr,
            embed_ind_hc_hr, embed_ind_hc_lr, embed_ind_lc_hr, embed_ind_lc_lr,
            dist_hr, dist_lr, dist_hr, dist_lr)


# speedup vs baseline: 2.3825x; 1.0014x over previous
import functools

import jax
import jax.numpy as jnp
from jax import lax
from jax.experimental import pallas as pl
from jax.experimental.pallas import tpu as pltpu
from jax.experimental.pallas import tpu_sc as plsc

_D = 64
_NE = 1024
_K = 64
_R = 1024
_NROW = 8192


_NW = 32
_BG = 4 * _NROW
_BPW = _BG // _NW
_CH = 128


def _sc_gather(table, idx2d):
    mesh = plsc.VectorSubcoreMesh(core_axis_name="c", subcore_axis_name="s")
    nch = _BPW // _CH

    @functools.partial(
        pl.kernel, mesh=mesh,
        out_type=jax.ShapeDtypeStruct((_BG, 128), jnp.float32),
        scratch_types=[
            pltpu.VMEM((nch, _CH), jnp.int32),
            pltpu.VMEM((_CH, 128), jnp.float32),
            pltpu.SemaphoreType.DMA,
        ],
    )
    def k(table_hbm, idx_hbm, out_hbm, idx_v, rows_v, sem):
        wid = lax.axis_index("s") * 2 + lax.axis_index("c")
        pltpu.sync_copy(idx_hbm.at[pl.ds(wid * nch, nch)], idx_v)
        for ci in range(nch):
            pltpu.async_copy(table_hbm.at[idx_v.at[ci]], rows_v, sem).wait()
            pltpu.sync_copy(rows_v, out_hbm.at[pl.ds(wid * _BPW + ci * _CH, _CH)])

    return k(table, idx2d)


def _body(x_ref, g_ref, e_ref, dist_ref, ihc_ref, ilc_ref):
    e = e_ref[...]
    x = x_ref[...]
    iota_e = jax.lax.broadcasted_iota(jnp.int32, (_R, _NE), 1)
    iota_k = jax.lax.broadcasted_iota(jnp.int32, (_R, _K), 1)

    r = jnp.sum(x * x, axis=1, keepdims=True)
    c = jnp.sum(e * e, axis=0, keepdims=True)
    m2 = jnp.dot(2.0 * x, e, preferred_element_type=jnp.float32)
    dist = (r - m2) + c
    dist_ref[...] = dist

    nd = -dist
    mv = jnp.max(nd, axis=1, keepdims=True)
    ihc = jnp.min(jnp.where(nd == mv, iota_e, _NE), axis=1, keepdims=True)
    ihc_ref[...] = ihc

    p = 1.0 / dist

    def step(k, carry):
        pm, sel, ps = carry
        v = jnp.max(pm, axis=1, keepdims=True)
        idx = jnp.min(jnp.where(pm == v, iota_e, _NE), axis=1, keepdims=True)
        pm = jnp.where(iota_e == idx, -jnp.inf, pm)
        sel = jnp.where(iota_k == k, idx, sel)
        ps = jnp.where(iota_k == k, v, ps)
        return pm, sel, ps

    _, sel, ps = jax.lax.fori_loop(
        0, _K, step,
        (p, jnp.zeros((_R, _K), jnp.int32), jnp.zeros((_R, _K), jnp.float32)))

    psn = ps / jnp.sum(ps, axis=1, keepdims=True)
    y = jnp.log(jnp.clip(psn, 1e-30, None)) + g_ref[...]
    ymax = jnp.max(y, axis=1, keepdims=True)
    samp = jnp.min(jnp.where(y == ymax, iota_k, _K), axis=1, keepdims=True)
    ilc = jnp.sum(jnp.where(iota_k == samp, sel, 0), axis=1, keepdims=True)
    ilc_ref[...] = ilc


@functools.partial(jax.jit, static_argnames=())
def _run(flat_hr, flat_lr, embed_lr, g_hr, g_lr):
    nb = _NROW // _R
    row_blk = lambda i: (i, 0)
    out_shapes = []
    out_specs = []
    for _ in range(2):
        out_shapes += [
            jax.ShapeDtypeStruct((_NROW, _NE), jnp.float32),
            jax.ShapeDtypeStruct((_NROW, 1), jnp.int32),
            jax.ShapeDtypeStruct((_NROW, 1), jnp.int32),
        ]
        out_specs += [
            pl.BlockSpec((_R, _NE), row_blk),
            pl.BlockSpec((_R, 1), row_blk),
            pl.BlockSpec((_R, 1), row_blk),
        ]

    def fused(xh_ref, xl_ref, gh_ref, gl_ref, e_ref,
              dh_ref, ihch_ref, ilch_ref,
              dl_ref, ihcl_ref, ilcl_ref):
        _body(xh_ref, gh_ref, e_ref, dh_ref, ihch_ref, ilch_ref)
        _body(xl_ref, gl_ref, e_ref, dl_ref, ihcl_ref, ilcl_ref)

    return pl.pallas_call(
        fused,
        grid=(nb,),
        in_specs=[
            pl.BlockSpec((_R, _D), row_blk),
            pl.BlockSpec((_R, _D), row_blk),
            pl.BlockSpec((_R, _K), row_blk),
            pl.BlockSpec((_R, _K), row_blk),
            pl.BlockSpec((_D, _NE), lambda i: (0, 0)),
        ],
        out_specs=out_specs,
        out_shape=out_shapes,
        compiler_params=pltpu.CompilerParams(
            dimension_semantics=("arbitrary",)),
    )(flat_hr, flat_lr, g_hr, g_lr, embed_lr)


def kernel(input_hr, input_lr, embed_lr):
    flat_hr = input_hr.reshape(-1, _D)
    flat_lr = input_lr.reshape(-1, _D)

    skey = jax.random.key(42)
    k1, k2 = jax.random.split(skey)
    g_lr = jax.random.gumbel(k1, (_NROW, _K), jnp.float32)
    g_hr = jax.random.gumbel(k2, (_NROW, _K), jnp.float32)

    (dist_hr, i_hc_hr, i_lc_hr,
     dist_lr, i_hc_lr, i_lc_lr) = _run(
        flat_hr, flat_lr, embed_lr, g_hr, g_lr)

    table = jnp.pad(embed_lr.T, ((0, 0), (0, 128 - _D)))
    idx2d = jnp.concatenate(
        [i_hc_hr, i_hc_lr, i_lc_hr, i_lc_lr], axis=0).reshape(_BG // _CH, _CH)
    q_all = _sc_gather(table, idx2d)[:, :_D]
    q_hc_hr = q_all[0 * _NROW:1 * _NROW]
    q_hc_lr = q_all[1 * _NROW:2 * _NROW]
    q_lc_hr = q_all[2 * _NROW:3 * _NROW]
    q_lc_lr = q_all[3 * _NROW:4 * _NROW]

    shp = input_hr.shape
    ind_shp = shp[:-1]

    def finish(q, inp):
        q = q.reshape(shp)
        d = q - inp
        return inp + d, jnp.mean(d * d)

    quantize_hc_hr, diff_hc_hr = finish(q_hc_hr, input_hr)
    quantize_hc_lr, diff_hc_lr = finish(q_hc_lr, input_lr)
    quantize_lc_hr, diff_lc_hr = finish(q_lc_hr, input_hr)
    quantize_lc_lr, diff_lc_lr = finish(q_lc_lr, input_lr)

    embed_ind_hc_hr = i_hc_hr.reshape(ind_shp)
    embed_ind_hc_lr = i_hc_lr.reshape(ind_shp)
    embed_ind_lc_hr = i_lc_hr.reshape(ind_shp)
    embed_ind_lc_lr = i_lc_lr.reshape(ind_shp)

    return (quantize_hc_hr, quantize_hc_lr, quantize_lc_hr, quantize_lc_lr,
            diff_hc_hr, diff_hc_lr, diff_lc_hr, diff_lc_lr,
            embed_ind_hc_hr, embed_ind_hc_lr, embed_ind_lc_hr, embed_ind_lc_lr,
            dist_hr, dist_lr, dist_hr, dist_lr)
